# baseline (speedup 1.0000x reference)
import functools

import jax
import jax.numpy as jnp
from jax import lax
from jax.experimental import pallas as pl
from jax.experimental.pallas import tpu as pltpu
from jax.experimental.pallas import tpu_sc as plsc

NC = 2
NS = 16
LANES = 16
IDX_CHUNK = 128


def _mf_body(n_factors, b_per_w, user_hbm, item_hbm, uf_hbm, if_hbm, out_hbm,
             uidx_v, iidx_v, urows_v, vrows_v, out_v, sem):
    wid = lax.axis_index("s") * NC + lax.axis_index("c")
    n_chunks = b_per_w // IDX_CHUNK

    pltpu.sync_copy(user_hbm.at[wid], uidx_v)
    pltpu.sync_copy(item_hbm.at[wid], iidx_v)

    copies = []
    for j in range(n_chunks):
        dst = pl.ds(j * IDX_CHUNK, IDX_CHUNK)
        copies.append(pltpu.async_copy(uf_hbm.at[uidx_v.at[j]],
                                       urows_v.at[dst], sem))
        copies.append(pltpu.async_copy(if_hbm.at[iidx_v.at[j]],
                                       vrows_v.at[dst], sem))
    for c in copies:
        c.wait()

    def group_body(g, carry):
        rows = g * LANES + lax.iota(jnp.int32, LANES)
        acc = jnp.zeros((LANES,), jnp.float32)
        for f in range(n_factors):
            cols = jnp.full((LANES,), f, jnp.int32)
            uu = plsc.load_gather(urows_v, [rows, cols])
            vv = plsc.load_gather(vrows_v, [rows, cols])
            acc = acc + uu * vv
        out_v[pl.ds(g * LANES, LANES)] = acc
        return carry

    lax.fori_loop(0, b_per_w // LANES, group_body, 0)
    pltpu.sync_copy(out_v, out_hbm.at[pl.ds(wid * b_per_w, b_per_w)])


def kernel(user, item, user_factors, item_factors):
    batch = user.shape[0]
    n_factors = user_factors.shape[1]
    nw = NC * NS
    b_per_w = batch // nw
    n_chunks = b_per_w // IDX_CHUNK

    user3 = user.astype(jnp.int32).reshape(nw, n_chunks, IDX_CHUNK)
    item3 = item.astype(jnp.int32).reshape(nw, n_chunks, IDX_CHUNK)

    mesh = plsc.VectorSubcoreMesh(core_axis_name="c", subcore_axis_name="s")
    run = pl.kernel(
        functools.partial(_mf_body, n_factors, b_per_w),
        out_type=jax.ShapeDtypeStruct((batch,), jnp.float32),
        mesh=mesh,
        scratch_types=[
            pltpu.VMEM((n_chunks, IDX_CHUNK), jnp.int32),
            pltpu.VMEM((n_chunks, IDX_CHUNK), jnp.int32),
            pltpu.VMEM((b_per_w, n_factors), jnp.float32),
            pltpu.VMEM((b_per_w, n_factors), jnp.float32),
            pltpu.VMEM((b_per_w,), jnp.float32),
            pltpu.SemaphoreType.DMA,
        ],
        compiler_params=pltpu.CompilerParams(
            needs_layout_passes=False, use_tc_tiling_on_sc=False),
    )
    return run(user3, item3, user_factors, item_factors)
